# baseline (speedup 1.0000x reference)
import functools

import jax
import jax.numpy as jnp
from jax import lax
from jax.experimental import pallas as pl
from jax.experimental.pallas import tpu as pltpu
from jax.experimental.pallas import tpu_sc as plsc

D = 128
M = 50000
KNBR = 4
B = 80
NBLK = M // B
NTASK = KNBR * NBLK
NC = 2
NS = 16
NW = NC * NS
NTPW = -(-NTASK // NW)
NTPW += NTPW % 2

_mesh = plsc.VectorSubcoreMesh(core_axis_name="c", subcore_axis_name="s")


@functools.partial(
    pl.kernel,
    out_type=jax.ShapeDtypeStruct((1, KNBR * (D // 2), 2 * M), jnp.float32),
    mesh=_mesh,
    compiler_params=pltpu.CompilerParams(
        use_tc_tiling_on_sc=False, needs_layout_passes=False
    ),
    scratch_types=[
        pltpu.VMEM((NTPW, B), jnp.int32),
        pltpu.VMEM((B, D), jnp.float32),
        pltpu.VMEM((B, D), jnp.float32),
        pltpu.VMEM((D, B + 1), jnp.float32),
        pltpu.VMEM((D, B + 1), jnp.float32),
        pltpu.SemaphoreType.DMA,
        pltpu.SemaphoreType.DMA,
        pltpu.SemaphoreType.DMA,
        pltpu.SemaphoreType.DMA,
    ],
)
def _gather_transpose(yd_hbm, idxw_hbm, out_hbm,
                      idx_all, rows0, rows1, t0, t1,
                      semr0, semr1, semt0, semt1):
    wid = lax.axis_index("s") * NC + lax.axis_index("c")
    last = jnp.where(wid < NTASK - (NTPW - 2) * NW, NTPW - 2, NTPW - 3)
    viota = lax.iota(jnp.int32, 16)
    srow = [
        ((16 * jf + viota) & 1) * (D // 2) + ((16 * jf + viota) >> 1)
        for jf in range(D // 16)
    ]
    rows = (rows0, rows1)
    ts = (t0, t1)
    semr = (semr0, semr1)
    semt = (semt0, semt1)

    pltpu.sync_copy(idxw_hbm.at[wid], idx_all)

    def start_gather(li, p):
        pltpu.async_copy(
            yd_hbm.at[idx_all.at[jnp.minimum(li, last)]], rows[p], semr[p]
        )

    def wait_rows(p):
        pltpu.make_async_copy(yd_hbm.at[pl.ds(0, B)], rows[p], semr[p]).wait()

    def wait_out(p):
        for s in range(2):
            pltpu.make_async_copy(
                ts[p].at[pl.ds(s * (D // 2), D // 2), pl.ds(0, B)],
                out_hbm.at[0, pl.ds(0, D // 2), pl.ds(s * M, B)],
                semt[p],
            ).wait()

    def transpose(p):
        @pl.loop(0, B)
        def _row(r):
            rc = jnp.full((16,), r, jnp.int32)
            for jf in range(D // 16):
                v = rows[p][r, pl.ds(16 * jf, 16)]
                plsc.store_scatter(ts[p], [srow[jf], rc], v)

    def start_out(li, p):
        t = wid + jnp.minimum(li, last) * NW
        kk = t // NBLK
        mm0 = (t % NBLK) * B
        for s in range(2):
            pltpu.async_copy(
                ts[p].at[pl.ds(s * (D // 2), D // 2), pl.ds(0, B)],
                out_hbm.at[0, pl.ds(kk * (D // 2), D // 2),
                           pl.ds(s * M + mm0, B)],
                semt[p],
            )

    def step(li, p, start_next=True, wait_t=True):
        if start_next:
            start_gather(li + 1, 1 - p)
        if wait_t:
            wait_out(p)
        wait_rows(p)
        transpose(p)
        start_out(li, p)

    start_gather(0, 0)
    step(0, 0, wait_t=False)
    step(1, 1, wait_t=False)

    @pl.loop(2, NTPW - 2, step=2)
    def _main(i):
        step(i, 0)
        step(i + 1, 1)

    step(NTPW - 2, 0)
    step(NTPW - 1, 1, start_next=False)
    wait_out(0)
    wait_out(1)


def kernel(y_patch, yd_patch, idx_k):
    del y_patch
    idxt = jnp.transpose(idx_k[0].astype(jnp.int32), (1, 0)).reshape(-1)
    tasks = jnp.pad(idxt.reshape(NTASK, B), ((0, NW * NTPW - NTASK), (0, 0)))
    idxw = jnp.transpose(tasks.reshape(NTPW, NW, B), (1, 0, 2))
    return _gather_transpose(yd_patch, idxw)


# speedup vs baseline: 1.2139x; 1.2139x over previous
import functools

import jax
import jax.numpy as jnp
from jax import lax
from jax.experimental import pallas as pl
from jax.experimental.pallas import tpu as pltpu
from jax.experimental.pallas import tpu_sc as plsc

D = 128
M = 50000
KNBR = 4
B = 80
NBLK = M // B
NTASK = KNBR * NBLK
NC = 2
NS = 16
NW = NC * NS
NTPW = -(-NTASK // NW)
NTPW += NTPW % 2
H = D // 2
DEW = H + 8

_mesh = plsc.VectorSubcoreMesh(core_axis_name="c", subcore_axis_name="s")


@functools.partial(
    pl.kernel,
    out_type=jax.ShapeDtypeStruct((1, 2 * M, KNBR * H), jnp.float32),
    mesh=_mesh,
    compiler_params=pltpu.CompilerParams(
        use_tc_tiling_on_sc=False, needs_layout_passes=False
    ),
    scratch_types=[
        pltpu.VMEM((NTPW, B), jnp.int32),
        pltpu.VMEM((B, D), jnp.float32),
        pltpu.VMEM((B, D), jnp.float32),
        pltpu.VMEM((B, 2 * DEW), jnp.float32),
        pltpu.VMEM((B, 2 * DEW), jnp.float32),
        pltpu.SemaphoreType.DMA,
        pltpu.SemaphoreType.DMA,
        pltpu.SemaphoreType.DMA,
        pltpu.SemaphoreType.DMA,
    ],
)
def _gather_deinterleave(yd_hbm, idxw_hbm, out_hbm,
                         idx_all, rows0, rows1, de0, de1,
                         semr0, semr1, semt0, semt1):
    wid = lax.axis_index("s") * NC + lax.axis_index("c")
    last = jnp.where(wid < NTASK - (NTPW - 2) * NW, NTPW - 2, NTPW - 3)
    viota = lax.iota(jnp.int32, 16)
    sidx = [(viota & 1) * DEW + 8 * u + (viota >> 1) for u in range(D // 16)]
    rows = (rows0, rows1)
    des = (de0, de1)
    semr = (semr0, semr1)
    semt = (semt0, semt1)

    pltpu.sync_copy(idxw_hbm.at[wid], idx_all)

    def start_gather(li, p):
        pltpu.async_copy(
            yd_hbm.at[idx_all.at[jnp.minimum(li, last)]], rows[p], semr[p]
        )

    def wait_rows(p):
        pltpu.make_async_copy(yd_hbm.at[pl.ds(0, B)], rows[p], semr[p]).wait()

    def wait_out(p):
        for s in range(2):
            pltpu.make_async_copy(
                des[p].at[:, pl.ds(s * DEW, H)],
                out_hbm.at[0, pl.ds(s * M, B), pl.ds(0, H)],
                semt[p],
            ).wait()

    def deinterleave(p):
        @pl.loop(0, B, unroll=2)
        def _row(c):
            drow = des[p].at[c]
            for u in range(D // 16):
                v = rows[p][c, pl.ds(16 * u, 16)]
                plsc.store_scatter(drow, [sidx[u]], v)

    def start_out(li, p):
        t = wid + jnp.minimum(li, last) * NW
        kk = t // NBLK
        mm0 = (t % NBLK) * B
        for s in range(2):
            pltpu.async_copy(
                des[p].at[:, pl.ds(s * DEW, H)],
                out_hbm.at[0, pl.ds(s * M + mm0, B), pl.ds(kk * H, H)],
                semt[p],
            )

    def step(li, p, start_next=True, wait_t=True):
        if start_next:
            start_gather(li + 1, 1 - p)
        if wait_t:
            wait_out(p)
        wait_rows(p)
        deinterleave(p)
        start_out(li, p)

    start_gather(0, 0)
    step(0, 0, wait_t=False)
    step(1, 1, wait_t=False)

    @pl.loop(2, NTPW - 2, step=2)
    def _main(i):
        step(i, 0)
        step(i + 1, 1)

    step(NTPW - 2, 0)
    step(NTPW - 1, 1, start_next=False)
    wait_out(0)
    wait_out(1)


def kernel(y_patch, yd_patch, idx_k):
    del y_patch
    idxt = jnp.transpose(idx_k[0].astype(jnp.int32), (1, 0)).reshape(-1)
    tasks = jnp.pad(idxt.reshape(NTASK, B), ((0, NW * NTPW - NTASK), (0, 0)))
    idxw = jnp.transpose(tasks.reshape(NTPW, NW, B), (1, 0, 2))
    p_out = _gather_deinterleave(yd_patch, idxw)
    return jnp.transpose(p_out, (0, 2, 1))


# speedup vs baseline: 2.2609x; 1.8625x over previous
import functools

import jax
import jax.numpy as jnp
from jax import lax
from jax.experimental import pallas as pl
from jax.experimental.pallas import tpu as pltpu
from jax.experimental.pallas import tpu_sc as plsc

D = 128
M = 50000
KNBR = 4
B = 80
NBLK = M // B
NPAIR = KNBR // 2
NTASK = NPAIR * NBLK
NC = 2
NS = 16
NW = NC * NS
NTPW = -(-NTASK // NW)
H = D // 2

_mesh = plsc.VectorSubcoreMesh(core_axis_name="c", subcore_axis_name="s")


@functools.partial(
    pl.kernel,
    out_type=jax.ShapeDtypeStruct((2 * M, KNBR * H), jnp.float32),
    mesh=_mesh,
    compiler_params=pltpu.CompilerParams(
        use_tc_tiling_on_sc=True, needs_layout_passes=False
    ),
    scratch_types=[
        pltpu.VMEM((2 * NTPW * D,), jnp.int32),
        pltpu.VMEM((B, D), jnp.float32),
        pltpu.VMEM((B, D), jnp.float32),
        pltpu.VMEM((B, D), jnp.float32),
        pltpu.VMEM((B, D), jnp.float32),
        pltpu.VMEM((B, 2 * D), jnp.float32),
        pltpu.VMEM((B, 2 * D), jnp.float32),
        pltpu.SemaphoreType.DMA,
        pltpu.SemaphoreType.DMA,
        pltpu.SemaphoreType.DMA,
        pltpu.SemaphoreType.DMA,
    ],
)
def _gather_deinterleave(yd_hbm, idxw_hbm, out_hbm,
                         idx_all, ra0, rb0, ra1, rb1, de0, de1,
                         semr0, semr1, semt0, semt1):
    wid = lax.axis_index("s") * NC + lax.axis_index("c")
    last = jnp.where(wid < NTASK - (NTPW - 1) * NW, NTPW - 1, NTPW - 2)
    viota = lax.iota(jnp.int32, 16)
    sidx = [[(viota & 1) * D + x * H + 8 * u + (viota >> 1)
             for u in range(D // 16)] for x in range(2)]
    rows = ((ra0, rb0), (ra1, rb1))
    des = (de0, de1)
    semr = (semr0, semr1)
    semt = (semt0, semt1)

    pltpu.sync_copy(
        idxw_hbm.at[pl.ds(wid * 2 * NTPW * D, 2 * NTPW * D)], idx_all
    )

    def start_gather(li, p):
        r2 = 2 * jnp.minimum(li, last)
        for x in range(2):
            pltpu.async_copy(
                yd_hbm.at[idx_all.at[pl.ds((r2 + x) * D, B)]],
                rows[p][x], semr[p],
            )

    def wait_rows(p):
        for x in range(2):
            pltpu.make_async_copy(
                yd_hbm.at[pl.ds(0, B)], rows[p][x], semr[p]
            ).wait()

    def wait_out(p):
        for s in range(2):
            pltpu.make_async_copy(
                des[p].at[:, pl.ds(s * D, D)],
                out_hbm.at[pl.ds(s * M, B), pl.ds(0, D)],
                semt[p],
            ).wait()

    def deinterleave(p):
        @pl.loop(0, B, unroll=2)
        def _row(c):
            rc = jnp.full((16,), c, jnp.int32)
            for x in range(2):
                src = rows[p][x]
                for u in range(D // 16):
                    v = src[c, pl.ds(16 * u, 16)]
                    plsc.store_scatter(des[p], [rc, sidx[x][u]], v)

    def start_out(li, p):
        t = wid + jnp.minimum(li, last) * NW
        q = t // NBLK
        mm0 = (t % NBLK) * B
        for s in range(2):
            pltpu.async_copy(
                des[p].at[:, pl.ds(s * D, D)],
                out_hbm.at[pl.ds(s * M + mm0, B), pl.ds(q * D, D)],
                semt[p],
            )

    def step(li, p, start_next=True, wait_t=True):
        if start_next:
            start_gather(li + 1, 1 - p)
        if wait_t:
            wait_out(p)
        wait_rows(p)
        deinterleave(p)
        start_out(li, p)

    start_gather(0, 0)
    step(0, 0, wait_t=False)
    step(1, 1, wait_t=False)

    @pl.loop(2, NTPW - 2, step=2)
    def _main(i):
        step(i, 0)
        step(i + 1, 1)

    step(NTPW - 2, 0)
    step(NTPW - 1, 1, start_next=False)
    wait_out(0)
    wait_out(1)


def kernel(y_patch, yd_patch, idx_k):
    del y_patch
    idxt = jnp.transpose(idx_k[0].astype(jnp.int32), (1, 0))
    blocks = idxt.reshape(NPAIR, 2, NBLK, B)
    tasks = jnp.transpose(blocks, (0, 2, 1, 3)).reshape(NTASK, 2, B)
    tasks = jnp.pad(tasks, ((0, NW * NTPW - NTASK), (0, 0), (0, D - B)))
    idxw = jnp.transpose(tasks.reshape(NTPW, NW, 2, D), (1, 0, 2, 3))
    idxw = idxw.reshape(NW * 2 * NTPW * D)
    p_out = _gather_deinterleave(yd_patch, idxw)
    return jnp.transpose(p_out, (1, 0))[None]


# speedup vs baseline: 4.6765x; 2.0684x over previous
import functools

import jax
import jax.numpy as jnp
from jax import lax
from jax.experimental import pallas as pl
from jax.experimental.pallas import tpu as pltpu
from jax.experimental.pallas import tpu_sc as plsc

D = 128
M = 50000
KNBR = 4
B = 80
NBLK = M // B
NPAIR = KNBR // 2
NTASK = NPAIR * NBLK
NC = 2
NS = 16
NW = NC * NS
NTPW = -(-NTASK // NW)
H = D // 2

_mesh = plsc.VectorSubcoreMesh(core_axis_name="c", subcore_axis_name="s")


@functools.partial(
    pl.kernel,
    out_type=jax.ShapeDtypeStruct((2 * M, KNBR * H), jnp.float32),
    mesh=_mesh,
    compiler_params=pltpu.CompilerParams(
        use_tc_tiling_on_sc=True, needs_layout_passes=False
    ),
    scratch_types=[
        pltpu.VMEM((2 * NTPW * D,), jnp.int32),
        pltpu.VMEM((B, D), jnp.float32),
        pltpu.VMEM((B, D), jnp.float32),
        pltpu.VMEM((B, D), jnp.float32),
        pltpu.VMEM((B, D), jnp.float32),
        pltpu.VMEM((B, 2 * D), jnp.float32),
        pltpu.VMEM((B, 2 * D), jnp.float32),
        pltpu.SemaphoreType.DMA,
        pltpu.SemaphoreType.DMA,
        pltpu.SemaphoreType.DMA,
        pltpu.SemaphoreType.DMA,
    ],
)
def _gather_deinterleave(yd_hbm, idxw_hbm, out_hbm,
                         idx_all, ra0, rb0, ra1, rb1, de0, de1,
                         semr0, semr1, semt0, semt1):
    wid = lax.axis_index("s") * NC + lax.axis_index("c")
    last = jnp.where(wid < NTASK - (NTPW - 1) * NW, NTPW - 1, NTPW - 2)
    viota = lax.iota(jnp.int32, 16)
    sidx = [[(viota & 1) * D + x * H + 8 * u + (viota >> 1)
             for u in range(D // 16)] for x in range(2)]
    rows = ((ra0, rb0), (ra1, rb1))
    des = (de0, de1)
    semr = (semr0, semr1)
    semt = (semt0, semt1)

    pltpu.sync_copy(
        idxw_hbm.at[pl.ds(wid * 2 * NTPW * D, 2 * NTPW * D)], idx_all
    )

    def start_gather(li, p):
        r2 = 2 * jnp.minimum(li, last)
        for x in range(2):
            pltpu.async_copy(
                yd_hbm.at[idx_all.at[pl.ds((r2 + x) * D, B)]],
                rows[p][x], semr[p],
            )

    def wait_rows(p):
        for x in range(2):
            pltpu.make_async_copy(
                yd_hbm.at[pl.ds(0, B)], rows[p][x], semr[p]
            ).wait()

    def wait_out(p):
        for s in range(2):
            pltpu.make_async_copy(
                des[p].at[:, pl.ds(s * D, D)],
                out_hbm.at[pl.ds(s * M, B), pl.ds(0, D)],
                semt[p],
            ).wait()

    def deinterleave(p):
        @plsc.parallel_loop(0, B, 1, unroll=2)
        def _row(c):
            rc = jnp.full((16,), c, jnp.int32)
            for x in range(2):
                src = rows[p][x]
                for u in range(D // 16):
                    v = src[c, pl.ds(16 * u, 16)]
                    plsc.store_scatter(des[p], [rc, sidx[x][u]], v)

    def start_out(li, p):
        t = wid + jnp.minimum(li, last) * NW
        q = t // NBLK
        mm0 = (t % NBLK) * B
        for s in range(2):
            pltpu.async_copy(
                des[p].at[:, pl.ds(s * D, D)],
                out_hbm.at[pl.ds(s * M + mm0, B), pl.ds(q * D, D)],
                semt[p],
            )

    def step(li, p, start_next=True, wait_t=True):
        if start_next:
            start_gather(li + 1, 1 - p)
        if wait_t:
            wait_out(p)
        wait_rows(p)
        deinterleave(p)
        start_out(li, p)

    start_gather(0, 0)
    step(0, 0, wait_t=False)
    step(1, 1, wait_t=False)

    @pl.loop(2, NTPW - 2, step=2)
    def _main(i):
        step(i, 0)
        step(i + 1, 1)

    step(NTPW - 2, 0)
    step(NTPW - 1, 1, start_next=False)
    wait_out(0)
    wait_out(1)


def kernel(y_patch, yd_patch, idx_k):
    del y_patch
    idxt = jnp.transpose(idx_k[0].astype(jnp.int32), (1, 0))
    blocks = idxt.reshape(NPAIR, 2, NBLK, B)
    tasks = jnp.transpose(blocks, (0, 2, 1, 3)).reshape(NTASK, 2, B)
    tasks = jnp.pad(tasks, ((0, NW * NTPW - NTASK), (0, 0), (0, D - B)))
    idxw = jnp.transpose(tasks.reshape(NTPW, NW, 2, D), (1, 0, 2, 3))
    idxw = idxw.reshape(NW * 2 * NTPW * D)
    p_out = _gather_deinterleave(yd_patch, idxw)
    return jnp.transpose(p_out, (1, 0))[None]
